# baseline (speedup 1.0000x reference)
import jax
import jax.numpy as jnp
from jax import lax
from jax.experimental import pallas as pl


ROWS = 4096
COLS = 20
VOCAB = 1000
VB = 40


def _onehot_block(x_ref, out_ref):
    c = pl.program_id(0)
    v0 = pl.program_id(1) * VB
    xv = x_ref[pl.ds(c, 1), :][:, None, :]
    iota = v0 + lax.broadcasted_iota(jnp.int32, (1, VB, ROWS), 1)
    out_ref[...] = (xv == iota).astype(jnp.int32)


def kernel(x):
    xt = x.astype(jnp.int32).T
    out_t = pl.pallas_call(
        _onehot_block,
        grid=(COLS, VOCAB // VB),
        in_specs=[pl.BlockSpec((COLS, ROWS), lambda c, v: (0, 0))],
        out_specs=pl.BlockSpec((1, VB, ROWS), lambda c, v: (c, v, 0)),
        out_shape=jax.ShapeDtypeStruct((COLS, VOCAB, ROWS), jnp.int32),
    )(xt)
    return jnp.transpose(out_t, (2, 0, 1))


# speedup vs baseline: 2.3689x; 2.3689x over previous
import jax
import jax.numpy as jnp
from jax import lax
from jax.experimental import pallas as pl


ROWS = 4096
COLS = 20
VOCAB = 1000
VB = 200


def _onehot_block(x_ref, out_ref):
    c = pl.program_id(0)
    v0 = pl.program_id(1) * VB
    xv = x_ref[pl.ds(c, 1), :][:, None, :]
    iota = v0 + lax.broadcasted_iota(jnp.int32, (1, VB, ROWS), 1)
    out_ref[...] = (xv == iota).astype(jnp.int32)


def kernel(x):
    xt = x.astype(jnp.int32).T
    out_t = pl.pallas_call(
        _onehot_block,
        grid=(COLS, VOCAB // VB),
        in_specs=[pl.BlockSpec((COLS, ROWS), lambda c, v: (0, 0))],
        out_specs=pl.BlockSpec((1, VB, ROWS), lambda c, v: (c, v, 0)),
        out_shape=jax.ShapeDtypeStruct((COLS, VOCAB, ROWS), jnp.int32),
    )(xt)
    return jnp.transpose(out_t, (2, 0, 1))
